# Initial kernel scaffold; baseline (speedup 1.0000x reference)
import jax
import jax.numpy as jnp
from jax.experimental import pallas as pl


def kernel(x, edge_index, W_proj, scoring_src, scoring_trg, W_skip, bias):
    raise NotImplementedError("write your pallas kernel here")


# speedup vs baseline: 1.0179x; 1.0179x over previous
import jax
import jax.numpy as jnp
from jax.experimental import pallas as pl

N, E, DIN, H, F = 10000, 320000, 128, 8, 16


def _epilogue_body(num_ref, denom_ref, skip_ref, bias_ref, out_ref):
    num = num_ref[...].reshape(-1, H, F)
    denom = denom_ref[...]
    att = num / (denom[..., None] + 1e-16)
    out = att.reshape(-1, H * F) + skip_ref[...] + bias_ref[...]
    out_ref[...] = jnp.where(out > 0, out, jnp.exp(jnp.minimum(out, 0.0)) - 1.0)


def kernel(x, edge_index, W_proj, scoring_src, scoring_trg, W_skip, bias):
    proj = (x @ W_proj).reshape(-1, H, F)
    scores_source = jnp.sum(proj * scoring_src, axis=-1)
    scores_target = jnp.sum(proj * scoring_trg, axis=-1)
    src = edge_index[0]
    trg = edge_index[1]
    scores_per_edge = jax.nn.leaky_relu(
        scores_source[src] + scores_target[trg], negative_slope=0.2)
    scores_per_edge = scores_per_edge - jnp.max(scores_per_edge)
    exp_scores = jnp.exp(scores_per_edge)
    denom = jax.ops.segment_sum(exp_scores, trg, num_segments=N)
    weighted = proj[src] * exp_scores[..., None]
    num = jax.ops.segment_sum(weighted, trg, num_segments=N)

    skip = x @ W_skip
    blk = 1000
    out = pl.pallas_call(
        _epilogue_body,
        grid=(N // blk,),
        in_specs=[
            pl.BlockSpec((blk, H * F), lambda i: (i, 0)),
            pl.BlockSpec((blk, H), lambda i: (i, 0)),
            pl.BlockSpec((blk, H * F), lambda i: (i, 0)),
            pl.BlockSpec((1, H * F), lambda i: (0, 0)),
        ],
        out_specs=pl.BlockSpec((blk, H * F), lambda i: (i, 0)),
        out_shape=jax.ShapeDtypeStruct((N, H * F), jnp.float32),
    )(num.reshape(N, H * F), denom, skip, bias.reshape(1, H * F))
    return (out, edge_index)


# speedup vs baseline: 3.3546x; 3.2957x over previous
import dataclasses
import functools

import jax
import jax.numpy as jnp
from jax import lax
from jax.experimental import pallas as pl
from jax.experimental.pallas import tpu as pltpu
from jax.experimental.pallas import tpu_sc as plsc

N, E, DIN, H, F = 10000, 320000, 128, 8, 16
HF = H * F

NC, NS = 2, 16
NW = NC * NS
NPAD = 10112
ACCW = NPAD * H
EPT = 10240
EPAD = EPT * NW
CHA = 64
NCHA = EPT // CHA
EPG = EPAD // 2
CHB = 128
NCHB = EPG // CHB
BLK = NPAD // 16


def _proj_scores_body(x_ref, wp_ref, sst_ref, proj_ref, sab_ref):
    p = jnp.dot(x_ref[...], wp_ref[...], preferred_element_type=jnp.float32)
    proj_ref[...] = p
    sab_ref[...] = jnp.dot(p, sst_ref[...], preferred_element_type=jnp.float32)


def _epilogue_body(num_ref, den_ref, x_ref, wsk_ref, bias_ref, r_ref, p_ref,
                   out_ref):
    num = jnp.zeros_like(out_ref)
    for hh in range(16):
        t = num_ref[0, hh] + num_ref[1, hh]
        num = num + jnp.dot(t, p_ref[hh], preferred_element_type=jnp.float32)
    den8 = den_ref[0]
    for w in range(1, NW):
        den8 = den8 + den_ref[w]
    denx = jnp.dot(den8, r_ref[...], preferred_element_type=jnp.float32)
    skip = jnp.dot(x_ref[...], wsk_ref[...], preferred_element_type=jnp.float32)
    out = num / (denx + 1e-16) + skip + bias_ref[...]
    out_ref[...] = jnp.where(out > 0, out, jnp.exp(jnp.minimum(out, 0.0)) - 1.0)


def _sc_scores_kernel(src_hbm, trg_hbm, sab_hbm, den_hbm, w_hbm,
                      srci, trgi, va, vb, wbuf, dacc):
    cid = lax.axis_index("c")
    sid = lax.axis_index("s")
    wid = cid * NS + sid
    i32 = jnp.int32
    lanes = lax.iota(i32, 16)
    perm_b = (lanes & 7) + 8
    mask8 = lanes < 8
    zl = jnp.zeros((16,), i32)

    @pl.loop(0, ACCW // 16)
    def _zero(i):
        dacc[pl.ds(i * 16, 16)] = jnp.zeros((16,), jnp.float32)

    @pl.loop(0, NCHA)
    def _chunk(i):
        base = wid * EPT + i * CHA
        pltpu.sync_copy(src_hbm.at[pl.ds(base, CHA)], srci.at[0])
        pltpu.sync_copy(trg_hbm.at[pl.ds(base, CHA)], trgi.at[0])
        pltpu.sync_copy(sab_hbm.at[srci.at[0]], va)
        pltpu.sync_copy(sab_hbm.at[trgi.at[0]], vb)

        @pl.loop(0, CHA)
        def _edge(c):
            cc = jnp.full((16,), c, i32)
            brot = plsc.load_gather(vb, [cc, perm_b])
            s = va[c, pl.ds(0, 16)] + brot
            s = jnp.maximum(s, 0.2 * s)
            w = jnp.exp(s)
            wbuf[c, :] = w
            trgv = plsc.load_gather(trgi, [zl, cc])
            plsc.addupdate_scatter(dacc, [trgv * 8 + lanes], w, mask=mask8)

        pltpu.sync_copy(wbuf, w_hbm.at[pl.ds(base, CHA), :])

    pltpu.sync_copy(dacc, den_hbm.at[wid])


def _sc_num_kernel(src_hbm, trg_hbm, w_hbm, proj_hbm, num_hbm,
                   srci, trgi, wbuf, rows, acc):
    cid = lax.axis_index("c")
    sid = lax.axis_index("s")
    wid = cid * NS + sid
    i32 = jnp.int32
    lanes = lax.iota(i32, 16)
    l7 = lanes & 7
    mask8 = lanes < 8
    zl = jnp.zeros((16,), i32)
    hh8 = sid * 8
    featv = l7 + hh8

    @pl.loop(0, ACCW // 16)
    def _zero(i):
        acc[pl.ds(i * 16, 16)] = jnp.zeros((16,), jnp.float32)

    headv = jnp.full((16,), 0, i32) + lax.div(sid, 2)

    @pl.loop(0, NCHB)
    def _chunk(i):
        base = cid * EPG + i * CHB
        pltpu.sync_copy(src_hbm.at[pl.ds(base, CHB)], srci.at[0])
        pltpu.sync_copy(trg_hbm.at[pl.ds(base, CHB)], trgi.at[0])
        pltpu.sync_copy(w_hbm.at[pl.ds(base, CHB), :], wbuf)
        pltpu.sync_copy(proj_hbm.at[srci.at[0]], rows)

        @pl.loop(0, CHB)
        def _edge(c):
            cc = jnp.full((16,), c, i32)
            wv = plsc.load_gather(wbuf, [cc, headv])
            v = plsc.load_gather(rows, [cc, featv])
            trgv = plsc.load_gather(trgi, [zl, cc])
            plsc.addupdate_scatter(acc, [trgv * 8 + lanes], v * wv, mask=mask8)

    pltpu.sync_copy(acc, num_hbm.at[wid])


def kernel(x, edge_index, W_proj, scoring_src, scoring_trg, W_skip, bias):
    f32 = jnp.float32
    i32 = jnp.int32
    eye8 = jnp.eye(H, dtype=f32)
    ss = (scoring_src.reshape(H, F)[:, :, None] * eye8[:, None, :]).reshape(HF, H)
    st = (scoring_trg.reshape(H, F)[:, :, None] * eye8[:, None, :]).reshape(HF, H)
    sst = jnp.concatenate([ss, st, jnp.zeros((HF, HF - 16), f32)], axis=1)
    r8 = jnp.kron(eye8, jnp.ones((1, F), f32))
    pmat = jnp.eye(HF, dtype=f32).reshape(16, 8, HF)

    xp = jnp.concatenate([x, jnp.zeros((NPAD - N, DIN), f32)], axis=0)
    epad = jnp.full((EPAD - E,), NPAD - 1, i32)
    srcp = jnp.concatenate([edge_index[0], epad])
    trgp = jnp.concatenate([edge_index[1], epad])

    proj, sab = pl.pallas_call(
        _proj_scores_body,
        grid=(NPAD // BLK,),
        in_specs=[
            pl.BlockSpec((BLK, DIN), lambda i: (i, 0)),
            pl.BlockSpec((DIN, HF), lambda i: (0, 0)),
            pl.BlockSpec((HF, HF), lambda i: (0, 0)),
        ],
        out_specs=[
            pl.BlockSpec((BLK, HF), lambda i: (i, 0)),
            pl.BlockSpec((BLK, HF), lambda i: (i, 0)),
        ],
        out_shape=[
            jax.ShapeDtypeStruct((NPAD, HF), f32),
            jax.ShapeDtypeStruct((NPAD, HF), f32),
        ],
    )(xp, W_proj, sst)

    mesh = plsc.VectorSubcoreMesh(core_axis_name="c", subcore_axis_name="s")
    cp = pltpu.CompilerParams()
    if "needs_layout_passes" in pltpu.CompilerParams.__dataclass_fields__:
        cp = dataclasses.replace(cp, needs_layout_passes=False)

    sc_scores = pl.kernel(
        _sc_scores_kernel,
        mesh=mesh,
        compiler_params=cp,
        out_type=[
            jax.ShapeDtypeStruct((NW, ACCW), f32),
            jax.ShapeDtypeStruct((EPAD, 16), f32),
        ],
        scratch_types=[
            pltpu.VMEM((1, CHA), i32),
            pltpu.VMEM((1, CHA), i32),
            pltpu.VMEM((CHA, HF), f32),
            pltpu.VMEM((CHA, HF), f32),
            pltpu.VMEM((CHA, 16), f32),
            pltpu.VMEM((ACCW,), f32),
        ],
    )
    dens, wedge = sc_scores(srcp, trgp, sab)

    sc_num = pl.kernel(
        _sc_num_kernel,
        mesh=mesh,
        compiler_params=cp,
        out_type=jax.ShapeDtypeStruct((NW, ACCW), f32),
        scratch_types=[
            pltpu.VMEM((1, CHB), i32),
            pltpu.VMEM((1, CHB), i32),
            pltpu.VMEM((CHB, 16), f32),
            pltpu.VMEM((CHB, HF), f32),
            pltpu.VMEM((ACCW,), f32),
        ],
    )
    nums = sc_num(srcp, trgp, wedge, proj)
    nums4 = nums.reshape(NC, 16, NPAD, H)
    dens3 = dens.reshape(NW, NPAD, H)

    out = pl.pallas_call(
        _epilogue_body,
        grid=(N // 200,),
        in_specs=[
            pl.BlockSpec((NC, 16, 200, H), lambda i: (0, 0, i, 0)),
            pl.BlockSpec((NW, 200, H), lambda i: (0, i, 0)),
            pl.BlockSpec((200, DIN), lambda i: (i, 0)),
            pl.BlockSpec((DIN, HF), lambda i: (0, 0)),
            pl.BlockSpec((1, HF), lambda i: (0, 0)),
            pl.BlockSpec((H, HF), lambda i: (0, 0)),
            pl.BlockSpec((16, H, HF), lambda i: (0, 0, 0)),
        ],
        out_specs=pl.BlockSpec((200, HF), lambda i: (i, 0)),
        out_shape=jax.ShapeDtypeStruct((N, HF), f32),
    )(nums4, dens3, x, W_skip, bias.reshape(1, HF), r8, pmat)
    return (out, edge_index)


# speedup vs baseline: 3.7105x; 1.1061x over previous
import dataclasses
import functools

import jax
import jax.numpy as jnp
from jax import lax
from jax.experimental import pallas as pl
from jax.experimental.pallas import tpu as pltpu
from jax.experimental.pallas import tpu_sc as plsc

N, E, DIN, H, F = 10000, 320000, 128, 8, 16
HF = H * F

NC, NS = 2, 16
NW = NC * NS
NPAD = 10112
ACCW = NPAD * H
EPT = 10240
EPAD = EPT * NW
CHA = 64
NCHA = EPT // CHA
EPG = EPAD // 2
CHB = 128
NCHB = EPG // CHB
BLK = NPAD // 16


def _proj_scores_body(x_ref, wp_ref, sst_ref, proj_ref, sab_ref):
    p = jnp.dot(x_ref[...], wp_ref[...], preferred_element_type=jnp.float32)
    proj_ref[...] = p
    sab_ref[...] = jnp.dot(p, sst_ref[...], preferred_element_type=jnp.float32)


def _epilogue_body(num_ref, den_ref, x_ref, wsk_ref, bias_ref, r_ref, p_ref,
                   out_ref):
    num = jnp.zeros_like(out_ref)
    for hh in range(16):
        t = num_ref[0, hh] + num_ref[1, hh]
        num = num + jnp.dot(t, p_ref[hh], preferred_element_type=jnp.float32)
    den8 = den_ref[0]
    for w in range(1, NW):
        den8 = den8 + den_ref[w]
    denx = jnp.dot(den8, r_ref[...], preferred_element_type=jnp.float32)
    skip = jnp.dot(x_ref[...], wsk_ref[...], preferred_element_type=jnp.float32)
    out = num / (denx + 1e-16) + skip + bias_ref[...]
    out_ref[...] = jnp.where(out > 0, out, jnp.exp(jnp.minimum(out, 0.0)) - 1.0)


def _sc_scores_kernel(src_hbm, trg_hbm, sab_hbm, den_hbm, w_hbm,
                      srci, trgi, va, vb, wbuf, dacc, dsem):
    cid = lax.axis_index("c")
    sid = lax.axis_index("s")
    wid = cid * NS + sid
    i32 = jnp.int32
    lanes = lax.iota(i32, 16)
    perm_b = (lanes & 7) + 8
    mask8 = lanes < 8
    zl = jnp.zeros((16,), i32)

    @pl.loop(0, ACCW // 16)
    def _zero(i):
        dacc[pl.ds(i * 16, 16)] = jnp.zeros((16,), jnp.float32)

    @pl.loop(0, NCHA)
    def _chunk(i):
        base = wid * EPT + i * CHA
        c1 = pltpu.async_copy(src_hbm.at[pl.ds(base, CHA)], srci.at[0], dsem)
        c2 = pltpu.async_copy(trg_hbm.at[pl.ds(base, CHA)], trgi.at[0], dsem)
        c1.wait()
        c2.wait()
        c3 = pltpu.async_copy(sab_hbm.at[srci.at[0]], va, dsem)
        c4 = pltpu.async_copy(sab_hbm.at[trgi.at[0]], vb, dsem)
        c3.wait()
        c4.wait()

        @pl.loop(0, CHA)
        def _edge(c):
            cc = jnp.full((16,), c, i32)
            brot = plsc.load_gather(vb, [cc, perm_b])
            s = va[c, pl.ds(0, 16)] + brot
            s = jnp.maximum(s, 0.2 * s)
            w = jnp.exp(s)
            wbuf[c, :] = w
            trgv = plsc.load_gather(trgi, [zl, cc])
            plsc.addupdate_scatter(dacc, [trgv * 8 + lanes], w, mask=mask8)

        pltpu.sync_copy(wbuf, w_hbm.at[pl.ds(base, CHA), :])

    pltpu.sync_copy(dacc, den_hbm.at[wid])


def _sc_num_kernel(src_hbm, trg_hbm, w_hbm, proj_hbm, num_hbm,
                   srci, trgi, wbuf, rows, acc, dsem):
    cid = lax.axis_index("c")
    sid = lax.axis_index("s")
    wid = cid * NS + sid
    i32 = jnp.int32
    lanes = lax.iota(i32, 16)
    l7 = lanes & 7
    mask8 = lanes < 8
    zl = jnp.zeros((16,), i32)
    hh8 = sid * 8
    featv = l7 + hh8

    @pl.loop(0, ACCW // 16)
    def _zero(i):
        acc[pl.ds(i * 16, 16)] = jnp.zeros((16,), jnp.float32)

    headv = jnp.full((16,), 0, i32) + lax.div(sid, 2)

    @pl.loop(0, NCHB)
    def _chunk(i):
        base = cid * EPG + i * CHB
        c1 = pltpu.async_copy(src_hbm.at[pl.ds(base, CHB)], srci.at[0], dsem)
        c2 = pltpu.async_copy(trg_hbm.at[pl.ds(base, CHB)], trgi.at[0], dsem)
        c3 = pltpu.async_copy(w_hbm.at[pl.ds(base, CHB), :], wbuf, dsem)
        c1.wait()
        c2.wait()
        c3.wait()
        pltpu.sync_copy(proj_hbm.at[srci.at[0]], rows)

        @pl.loop(0, CHB)
        def _edge(c):
            cc = jnp.full((16,), c, i32)
            wv = plsc.load_gather(wbuf, [cc, headv])
            v = plsc.load_gather(rows, [cc, featv])
            trgv = plsc.load_gather(trgi, [zl, cc])
            plsc.addupdate_scatter(acc, [trgv * 8 + lanes], v * wv, mask=mask8)

    pltpu.sync_copy(acc, num_hbm.at[wid])


def kernel(x, edge_index, W_proj, scoring_src, scoring_trg, W_skip, bias):
    f32 = jnp.float32
    i32 = jnp.int32
    eye8 = jnp.eye(H, dtype=f32)
    ss = (scoring_src.reshape(H, F)[:, :, None] * eye8[:, None, :]).reshape(HF, H)
    st = (scoring_trg.reshape(H, F)[:, :, None] * eye8[:, None, :]).reshape(HF, H)
    sst = jnp.concatenate([ss, st, jnp.zeros((HF, HF - 16), f32)], axis=1)
    r8 = jnp.kron(eye8, jnp.ones((1, F), f32))
    pmat = jnp.eye(HF, dtype=f32).reshape(16, 8, HF)

    xp = jnp.concatenate([x, jnp.zeros((NPAD - N, DIN), f32)], axis=0)
    epad = jnp.full((EPAD - E,), NPAD - 1, i32)
    srcp = jnp.concatenate([edge_index[0], epad])
    trgp = jnp.concatenate([edge_index[1], epad])

    proj, sab = pl.pallas_call(
        _proj_scores_body,
        grid=(NPAD // BLK,),
        in_specs=[
            pl.BlockSpec((BLK, DIN), lambda i: (i, 0)),
            pl.BlockSpec((DIN, HF), lambda i: (0, 0)),
            pl.BlockSpec((HF, HF), lambda i: (0, 0)),
        ],
        out_specs=[
            pl.BlockSpec((BLK, HF), lambda i: (i, 0)),
            pl.BlockSpec((BLK, HF), lambda i: (i, 0)),
        ],
        out_shape=[
            jax.ShapeDtypeStruct((NPAD, HF), f32),
            jax.ShapeDtypeStruct((NPAD, HF), f32),
        ],
    )(xp, W_proj, sst)

    mesh = plsc.VectorSubcoreMesh(core_axis_name="c", subcore_axis_name="s")
    cp = pltpu.CompilerParams()
    if "needs_layout_passes" in pltpu.CompilerParams.__dataclass_fields__:
        cp = dataclasses.replace(cp, needs_layout_passes=False)

    sc_scores = pl.kernel(
        _sc_scores_kernel,
        mesh=mesh,
        compiler_params=cp,
        out_type=[
            jax.ShapeDtypeStruct((NW, ACCW), f32),
            jax.ShapeDtypeStruct((EPAD, 16), f32),
        ],
        scratch_types=[
            pltpu.VMEM((1, CHA), i32),
            pltpu.VMEM((1, CHA), i32),
            pltpu.VMEM((CHA, HF), f32),
            pltpu.VMEM((CHA, HF), f32),
            pltpu.VMEM((CHA, 16), f32),
            pltpu.VMEM((ACCW,), f32),
            pltpu.SemaphoreType.DMA,
        ],
    )
    dens, wedge = sc_scores(srcp, trgp, sab)

    sc_num = pl.kernel(
        _sc_num_kernel,
        mesh=mesh,
        compiler_params=cp,
        out_type=jax.ShapeDtypeStruct((NW, ACCW), f32),
        scratch_types=[
            pltpu.VMEM((1, CHB), i32),
            pltpu.VMEM((1, CHB), i32),
            pltpu.VMEM((CHB, 16), f32),
            pltpu.VMEM((CHB, HF), f32),
            pltpu.VMEM((ACCW,), f32),
            pltpu.SemaphoreType.DMA,
        ],
    )
    nums = sc_num(srcp, trgp, wedge, proj)
    nums4 = nums.reshape(NC, 16, NPAD, H)
    dens3 = dens.reshape(NW, NPAD, H)

    out = pl.pallas_call(
        _epilogue_body,
        grid=(N // 200,),
        in_specs=[
            pl.BlockSpec((NC, 16, 200, H), lambda i: (0, 0, i, 0)),
            pl.BlockSpec((NW, 200, H), lambda i: (0, i, 0)),
            pl.BlockSpec((200, DIN), lambda i: (i, 0)),
            pl.BlockSpec((DIN, HF), lambda i: (0, 0)),
            pl.BlockSpec((1, HF), lambda i: (0, 0)),
            pl.BlockSpec((H, HF), lambda i: (0, 0)),
            pl.BlockSpec((16, H, HF), lambda i: (0, 0, 0)),
        ],
        out_specs=pl.BlockSpec((200, HF), lambda i: (i, 0)),
        out_shape=jax.ShapeDtypeStruct((N, HF), f32),
    )(nums4, dens3, x, W_skip, bias.reshape(1, HF), r8, pmat)
    return (out, edge_index)
